# baseline (speedup 1.0000x reference)
import jax
import jax.numpy as jnp
from jax.experimental import pallas as pl
from jax.experimental.pallas import tpu as pltpu

_HIGHEST = jax.lax.Precision.HIGHEST
_EPS = 1e-5


def _pass_a(qa_ref, q_ref, of_ref, pa_ref, pf_ref, w0c_ref, w0f_ref, w0i_ref,
            b0_ref, y0_ref, st_ref):
    d2 = jnp.dot(qa_ref[...], pa_ref[...],
                 preferred_element_type=jnp.float32)
    m1 = jnp.min(d2, axis=1, keepdims=True)
    t = jnp.where(d2 == m1, jnp.inf, d2)
    m2 = jnp.min(t, axis=1, keepdims=True)
    t = jnp.where(t == m2, jnp.inf, t)
    m3 = jnp.min(t, axis=1, keepdims=True)
    w = jnp.where(d2 <= m3, 1.0 / jnp.maximum(d2, 1e-12), 0.0)
    wsum = (1.0 / jnp.maximum(m1, 1e-12) + 1.0 / jnp.maximum(m2, 1e-12)
            + 1.0 / jnp.maximum(m3, 1e-12))
    interp = jnp.dot(w, pf_ref[...],
                     preferred_element_type=jnp.float32) / wsum
    y0 = (jnp.dot(q_ref[...], w0c_ref[...], precision=_HIGHEST,
                  preferred_element_type=jnp.float32)
          + jnp.dot(of_ref[...], w0f_ref[...], precision=_HIGHEST,
                    preferred_element_type=jnp.float32)
          + jnp.dot(interp, w0i_ref[...], precision=_HIGHEST,
                    preferred_element_type=jnp.float32)
          + b0_ref[...])
    y0_ref[...] = y0
    st_ref[0, 0:1, :] = jnp.sum(y0, axis=0, keepdims=True)
    st_ref[0, 1:2, :] = jnp.sum(y0 * y0, axis=0, keepdims=True)


def _make_pass_bc(n_rows, with_matmul):
    inv_n = 1.0 / n_rows

    def _stats(st_ref, g_ref, be_ref):
        part = jnp.sum(st_ref[...], axis=0)
        mean = part[0:1, :] * inv_n
        var = part[1:2, :] * inv_n - mean * mean
        scale = g_ref[...] * jax.lax.rsqrt(var + _EPS)
        shift = be_ref[...] - mean * scale
        return scale, shift

    def _pass_b(y_ref, st_ref, g_ref, be_ref, w1_ref, b1_ref, o_ref, st1_ref):
        scale, shift = _stats(st_ref, g_ref, be_ref)
        h = jnp.maximum(y_ref[...] * scale + shift, 0.0)
        y1 = jnp.dot(h, w1_ref[...], precision=_HIGHEST,
                     preferred_element_type=jnp.float32) + b1_ref[...]
        o_ref[...] = y1
        st1_ref[0, 0:1, :] = jnp.sum(y1, axis=0, keepdims=True)
        st1_ref[0, 1:2, :] = jnp.sum(y1 * y1, axis=0, keepdims=True)

    def _pass_c(y_ref, st_ref, g_ref, be_ref, o_ref):
        scale, shift = _stats(st_ref, g_ref, be_ref)
        o_ref[...] = jnp.maximum(y_ref[...] * scale + shift, 0.0)

    return _pass_b if with_matmul else _pass_c


def kernel(prop_coords, prop_feats, orig_coords, orig_feats,
           W0, b0, g0, be0, W1, b1, g1, be1):
    n_l, _ = prop_coords.shape
    n_m = orig_coords.shape[0]
    f1 = prop_feats.shape[1]
    f2 = orig_feats.shape[1]
    h = W0.shape[1]
    tm = 512
    grid = n_m // tm
    f32 = jnp.float32

    qn2 = jnp.sum(orig_coords * orig_coords, axis=1, keepdims=True)
    pn2 = jnp.sum(prop_coords * prop_coords, axis=1, keepdims=True)
    ones_q = jnp.ones((n_m, 1), f32)
    zeros_q = jnp.zeros((n_m, 3), f32)
    qa = jnp.concatenate([orig_coords, qn2, ones_q, zeros_q], axis=1)
    ones_p = jnp.ones((n_l, 1), f32)
    zeros_p = jnp.zeros((n_l, 3), f32)
    pa = jnp.concatenate([-2.0 * prop_coords, ones_p, pn2, zeros_p],
                         axis=1).T

    def _split3(x):
        hi = jax.lax.reduce_precision(x, 8, 7)
        r = x - hi
        mid = jax.lax.reduce_precision(r, 8, 7)
        lo = jax.lax.reduce_precision(r - mid, 8, 7)
        return hi, mid, lo

    qh, qm, ql = _split3(qa)
    ph, pm, plo = _split3(pa)
    zq = jnp.zeros((n_m, 80), f32)
    zp = jnp.zeros((80, n_l), f32)
    qs = jnp.concatenate([qh, qh, qm, qh, ql, qm, zq], axis=1)
    ps = jnp.concatenate([ph, pm, ph, plo, ph, pm, zp], axis=0)
    w0c = jnp.pad(W0[:3], ((0, 5), (0, 0)))
    w0f = W0[3:3 + f2]
    w0i = W0[3 + f2:]
    b0r = b0.reshape(1, h)
    b1r = b1.reshape(1, h)
    g0r = g0.reshape(1, h)
    be0r = be0.reshape(1, h)
    g1r = g1.reshape(1, h)
    be1r = be1.reshape(1, h)

    row_spec = lambda w: pl.BlockSpec((tm, w), lambda i: (i, 0))
    full = lambda shape: pl.BlockSpec(shape, lambda i: (0,) * len(shape))
    st_out_spec = pl.BlockSpec((1, 8, h), lambda i: (i, 0, 0))
    st_in_spec = pl.BlockSpec((grid, 8, h), lambda i: (0, 0, 0))
    st_shape = jax.ShapeDtypeStruct((grid, 8, h), f32)
    params = pltpu.CompilerParams(dimension_semantics=("parallel",))

    y0, st0 = pl.pallas_call(
        _pass_a,
        grid=(grid,),
        in_specs=[row_spec(128), row_spec(8), row_spec(f2), full((128, n_l)),
                  full((n_l, f1)), full((8, h)), full((f2, h)), full((f1, h)),
                  full((1, h))],
        out_specs=[row_spec(h), st_out_spec],
        out_shape=[jax.ShapeDtypeStruct((n_m, h), f32), st_shape],
        compiler_params=params,
    )(qs, qa, orig_feats, ps, prop_feats, w0c, w0f, w0i, b0r)

    y1, st1 = pl.pallas_call(
        _make_pass_bc(n_m, True),
        grid=(grid,),
        in_specs=[row_spec(h), st_in_spec, full((1, h)), full((1, h)),
                  full((h, h)), full((1, h))],
        out_specs=[row_spec(h), st_out_spec],
        out_shape=[jax.ShapeDtypeStruct((n_m, h), f32), st_shape],
        compiler_params=params,
    )(y0, st0, g0r, be0r, W1, b1r)

    out = pl.pallas_call(
        _make_pass_bc(n_m, False),
        grid=(grid,),
        in_specs=[row_spec(h), st_in_spec, full((1, h)), full((1, h))],
        out_specs=row_spec(h),
        out_shape=jax.ShapeDtypeStruct((n_m, h), f32),
        compiler_params=params,
    )(y1, st1, g1r, be1r)

    return out


# speedup vs baseline: 1.1912x; 1.1912x over previous
import jax
import jax.numpy as jnp
from jax.experimental import pallas as pl
from jax.experimental.pallas import tpu as pltpu

_HIGHEST = jax.lax.Precision.HIGHEST
_EPS = 1e-5


def _pass_a(qa_ref, q_ref, of_ref, pa_ref, pf_ref, w0c_ref, w0f_ref, w0i_ref,
            b0_ref, y0_ref, st_ref):
    d2 = jnp.dot(qa_ref[...], pa_ref[...],
                 preferred_element_type=jnp.float32)
    m1 = jnp.min(d2, axis=1, keepdims=True)
    t = jnp.where(d2 == m1, jnp.inf, d2)
    m2 = jnp.min(t, axis=1, keepdims=True)
    t = jnp.where(t == m2, jnp.inf, t)
    m3 = jnp.min(t, axis=1, keepdims=True)
    w = jnp.where(d2 <= m3, 1.0 / jnp.maximum(d2, 1e-12), 0.0)
    wsum = (1.0 / jnp.maximum(m1, 1e-12) + 1.0 / jnp.maximum(m2, 1e-12)
            + 1.0 / jnp.maximum(m3, 1e-12))
    interp = jnp.dot(w, pf_ref[...],
                     preferred_element_type=jnp.float32) / wsum
    y0 = (jnp.dot(q_ref[...], w0c_ref[...], precision=_HIGHEST,
                  preferred_element_type=jnp.float32)
          + jnp.dot(of_ref[...], w0f_ref[...], precision=_HIGHEST,
                    preferred_element_type=jnp.float32)
          + jnp.dot(interp, w0i_ref[...], precision=_HIGHEST,
                    preferred_element_type=jnp.float32)
          + b0_ref[...])
    y0_ref[...] = y0
    st_ref[0, 0:1, :] = jnp.sum(y0, axis=0, keepdims=True)
    st_ref[0, 1:2, :] = jnp.sum(y0 * y0, axis=0, keepdims=True)


def _make_pass_bc(n_rows, with_matmul):
    inv_n = 1.0 / n_rows

    def _stats(st_ref, g_ref, be_ref):
        part = jnp.sum(st_ref[...], axis=0)
        mean = part[0:1, :] * inv_n
        var = part[1:2, :] * inv_n - mean * mean
        scale = g_ref[...] * jax.lax.rsqrt(var + _EPS)
        shift = be_ref[...] - mean * scale
        return scale, shift

    def _pass_b(y_ref, st_ref, g_ref, be_ref, w1_ref, b1_ref, o_ref, st1_ref):
        scale, shift = _stats(st_ref, g_ref, be_ref)
        h = jnp.maximum(y_ref[...] * scale + shift, 0.0)
        y1 = jnp.dot(h, w1_ref[...], precision=_HIGHEST,
                     preferred_element_type=jnp.float32) + b1_ref[...]
        o_ref[...] = y1
        st1_ref[0, 0:1, :] = jnp.sum(y1, axis=0, keepdims=True)
        st1_ref[0, 1:2, :] = jnp.sum(y1 * y1, axis=0, keepdims=True)

    def _pass_c(y_ref, st_ref, g_ref, be_ref, o_ref):
        scale, shift = _stats(st_ref, g_ref, be_ref)
        o_ref[...] = jnp.maximum(y_ref[...] * scale + shift, 0.0)

    return _pass_b if with_matmul else _pass_c


def kernel(prop_coords, prop_feats, orig_coords, orig_feats,
           W0, b0, g0, be0, W1, b1, g1, be1):
    n_l, _ = prop_coords.shape
    n_m = orig_coords.shape[0]
    f1 = prop_feats.shape[1]
    f2 = orig_feats.shape[1]
    h = W0.shape[1]
    tm = 512
    grid = n_m // tm
    f32 = jnp.float32

    qn2 = jnp.sum(orig_coords * orig_coords, axis=1, keepdims=True)
    pn2 = jnp.sum(prop_coords * prop_coords, axis=1, keepdims=True)
    ones_q = jnp.ones((n_m, 1), f32)
    zeros_q = jnp.zeros((n_m, 3), f32)
    qa = jnp.concatenate([orig_coords, qn2, ones_q, zeros_q], axis=1)
    ones_p = jnp.ones((n_l, 1), f32)
    zeros_p = jnp.zeros((n_l, 3), f32)
    pa = jnp.concatenate([-2.0 * prop_coords, ones_p, pn2, zeros_p],
                         axis=1).T

    def _split3(x):
        hi = jax.lax.reduce_precision(x, 8, 7)
        r = x - hi
        mid = jax.lax.reduce_precision(r, 8, 7)
        lo = jax.lax.reduce_precision(r - mid, 8, 7)
        return hi, mid, lo

    qh, qm, ql = _split3(qa)
    ph, pm, plo = _split3(pa)
    qs = jnp.concatenate([qh, qh, qm, qh, ql, qm], axis=1)
    ps = jnp.concatenate([ph, pm, ph, plo, ph, pm], axis=0)
    w0c = jnp.pad(W0[:3], ((0, 5), (0, 0)))
    w0f = W0[3:3 + f2]
    w0i = W0[3 + f2:]
    b0r = b0.reshape(1, h)
    b1r = b1.reshape(1, h)
    g0r = g0.reshape(1, h)
    be0r = be0.reshape(1, h)
    g1r = g1.reshape(1, h)
    be1r = be1.reshape(1, h)

    row_spec = lambda w: pl.BlockSpec((tm, w), lambda i: (i, 0))
    full = lambda shape: pl.BlockSpec(shape, lambda i: (0,) * len(shape))
    st_out_spec = pl.BlockSpec((1, 8, h), lambda i: (i, 0, 0))
    st_in_spec = pl.BlockSpec((grid, 8, h), lambda i: (0, 0, 0))
    st_shape = jax.ShapeDtypeStruct((grid, 8, h), f32)
    params = pltpu.CompilerParams(dimension_semantics=("parallel",))

    y0, st0 = pl.pallas_call(
        _pass_a,
        grid=(grid,),
        in_specs=[row_spec(48), row_spec(8), row_spec(f2), full((48, n_l)),
                  full((n_l, f1)), full((8, h)), full((f2, h)), full((f1, h)),
                  full((1, h))],
        out_specs=[row_spec(h), st_out_spec],
        out_shape=[jax.ShapeDtypeStruct((n_m, h), f32), st_shape],
        compiler_params=params,
    )(qs, qa, orig_feats, ps, prop_feats, w0c, w0f, w0i, b0r)

    y1, st1 = pl.pallas_call(
        _make_pass_bc(n_m, True),
        grid=(grid,),
        in_specs=[row_spec(h), st_in_spec, full((1, h)), full((1, h)),
                  full((h, h)), full((1, h))],
        out_specs=[row_spec(h), st_out_spec],
        out_shape=[jax.ShapeDtypeStruct((n_m, h), f32), st_shape],
        compiler_params=params,
    )(y0, st0, g0r, be0r, W1, b1r)

    out = pl.pallas_call(
        _make_pass_bc(n_m, False),
        grid=(grid,),
        in_specs=[row_spec(h), st_in_spec, full((1, h)), full((1, h))],
        out_specs=row_spec(h),
        out_shape=jax.ShapeDtypeStruct((n_m, h), f32),
        compiler_params=params,
    )(y1, st1, g1r, be1r)

    return out


# speedup vs baseline: 1.2020x; 1.0091x over previous
import jax
import jax.numpy as jnp
from jax.experimental import pallas as pl
from jax.experimental.pallas import tpu as pltpu

_HIGHEST = jax.lax.Precision.HIGHEST
_EPS = 1e-5


def _pass_a(qa_ref, q_ref, of_ref, pa_ref, pf_ref, w0c_ref, w0f_ref, w0i_ref,
            b0_ref, y0_ref, st_ref):
    d2 = jnp.dot(qa_ref[...], pa_ref[...],
                 preferred_element_type=jnp.float32)
    m1 = jnp.min(d2, axis=1, keepdims=True)
    t = jnp.where(d2 == m1, jnp.inf, d2)
    m2 = jnp.min(t, axis=1, keepdims=True)
    t = jnp.where(t == m2, jnp.inf, t)
    m3 = jnp.min(t, axis=1, keepdims=True)
    w = jnp.where(d2 <= m3, 1.0 / jnp.maximum(d2, 1e-12), 0.0)
    wsum = (1.0 / jnp.maximum(m1, 1e-12) + 1.0 / jnp.maximum(m2, 1e-12)
            + 1.0 / jnp.maximum(m3, 1e-12))
    interp = jnp.dot(w, pf_ref[...],
                     preferred_element_type=jnp.float32) / wsum
    y0 = (jnp.dot(q_ref[...], w0c_ref[...], precision=_HIGHEST,
                  preferred_element_type=jnp.float32)
          + jnp.dot(of_ref[...], w0f_ref[...], precision=_HIGHEST,
                    preferred_element_type=jnp.float32)
          + jnp.dot(interp, w0i_ref[...], precision=_HIGHEST,
                    preferred_element_type=jnp.float32)
          + b0_ref[...])
    y0_ref[...] = y0
    st_ref[0, 0:1, :] = jnp.sum(y0, axis=0, keepdims=True)
    st_ref[0, 1:2, :] = jnp.sum(y0 * y0, axis=0, keepdims=True)


def _make_pass_bc(n_rows, with_matmul):
    inv_n = 1.0 / n_rows

    def _stats(st_ref, g_ref, be_ref):
        part = jnp.sum(st_ref[...], axis=0)
        mean = part[0:1, :] * inv_n
        var = part[1:2, :] * inv_n - mean * mean
        scale = g_ref[...] * jax.lax.rsqrt(var + _EPS)
        shift = be_ref[...] - mean * scale
        return scale, shift

    def _pass_b(y_ref, st_ref, g_ref, be_ref, w1_ref, b1_ref, o_ref, st1_ref):
        scale, shift = _stats(st_ref, g_ref, be_ref)
        h = jnp.maximum(y_ref[...] * scale + shift, 0.0)
        y1 = jnp.dot(h, w1_ref[...], precision=_HIGHEST,
                     preferred_element_type=jnp.float32) + b1_ref[...]
        o_ref[...] = y1
        st1_ref[0, 0:1, :] = jnp.sum(y1, axis=0, keepdims=True)
        st1_ref[0, 1:2, :] = jnp.sum(y1 * y1, axis=0, keepdims=True)

    def _pass_c(y_ref, st_ref, g_ref, be_ref, o_ref):
        scale, shift = _stats(st_ref, g_ref, be_ref)
        o_ref[...] = jnp.maximum(y_ref[...] * scale + shift, 0.0)

    return _pass_b if with_matmul else _pass_c


def kernel(prop_coords, prop_feats, orig_coords, orig_feats,
           W0, b0, g0, be0, W1, b1, g1, be1):
    n_l, _ = prop_coords.shape
    n_m = orig_coords.shape[0]
    f1 = prop_feats.shape[1]
    f2 = orig_feats.shape[1]
    h = W0.shape[1]
    tm = 512
    grid = n_m // tm
    f32 = jnp.float32

    qn2 = jnp.sum(orig_coords * orig_coords, axis=1, keepdims=True)
    pn2 = jnp.sum(prop_coords * prop_coords, axis=1, keepdims=True)
    ones_q = jnp.ones((n_m, 1), f32)
    zeros_q = jnp.zeros((n_m, 3), f32)
    qa = jnp.concatenate([orig_coords, qn2, ones_q, zeros_q], axis=1)
    ones_p = jnp.ones((n_l, 1), f32)
    zeros_p = jnp.zeros((n_l, 3), f32)
    pa = jnp.concatenate([-2.0 * prop_coords, ones_p, pn2, zeros_p],
                         axis=1).T

    def _split3(x):
        hi = jax.lax.reduce_precision(x, 8, 7)
        r = x - hi
        mid = jax.lax.reduce_precision(r, 8, 7)
        lo = jax.lax.reduce_precision(r - mid, 8, 7)
        return hi, mid, lo

    qh, qm, ql = _split3(qa)
    ph, pm, plo = _split3(pa)
    qs = jnp.concatenate([qh, qh, qm, qh, ql, qm],
                         axis=1).astype(jnp.bfloat16)
    ps = jnp.concatenate([ph, pm, ph, plo, ph, pm],
                         axis=0).astype(jnp.bfloat16)
    w0c = jnp.pad(W0[:3], ((0, 5), (0, 0)))
    w0f = W0[3:3 + f2]
    w0i = W0[3 + f2:]
    b0r = b0.reshape(1, h)
    b1r = b1.reshape(1, h)
    g0r = g0.reshape(1, h)
    be0r = be0.reshape(1, h)
    g1r = g1.reshape(1, h)
    be1r = be1.reshape(1, h)

    row_spec = lambda w: pl.BlockSpec((tm, w), lambda i: (i, 0))
    full = lambda shape: pl.BlockSpec(shape, lambda i: (0,) * len(shape))
    st_out_spec = pl.BlockSpec((1, 8, h), lambda i: (i, 0, 0))
    st_in_spec = pl.BlockSpec((grid, 8, h), lambda i: (0, 0, 0))
    st_shape = jax.ShapeDtypeStruct((grid, 8, h), f32)
    params = pltpu.CompilerParams(dimension_semantics=("parallel",))

    y0, st0 = pl.pallas_call(
        _pass_a,
        grid=(grid,),
        in_specs=[row_spec(48), row_spec(8), row_spec(f2), full((48, n_l)),
                  full((n_l, f1)), full((8, h)), full((f2, h)), full((f1, h)),
                  full((1, h))],
        out_specs=[row_spec(h), st_out_spec],
        out_shape=[jax.ShapeDtypeStruct((n_m, h), f32), st_shape],
        compiler_params=params,
    )(qs, qa, orig_feats, ps, prop_feats, w0c, w0f, w0i, b0r)

    y1, st1 = pl.pallas_call(
        _make_pass_bc(n_m, True),
        grid=(grid,),
        in_specs=[row_spec(h), st_in_spec, full((1, h)), full((1, h)),
                  full((h, h)), full((1, h))],
        out_specs=[row_spec(h), st_out_spec],
        out_shape=[jax.ShapeDtypeStruct((n_m, h), f32), st_shape],
        compiler_params=params,
    )(y0, st0, g0r, be0r, W1, b1r)

    out = pl.pallas_call(
        _make_pass_bc(n_m, False),
        grid=(grid,),
        in_specs=[row_spec(h), st_in_spec, full((1, h)), full((1, h))],
        out_specs=row_spec(h),
        out_shape=jax.ShapeDtypeStruct((n_m, h), f32),
        compiler_params=params,
    )(y1, st1, g1r, be1r)

    return out


# speedup vs baseline: 1.3250x; 1.1023x over previous
import jax
import jax.numpy as jnp
from jax.experimental import pallas as pl
from jax.experimental.pallas import tpu as pltpu

_HIGHEST = jax.lax.Precision.HIGHEST
_EPS = 1e-5


def _pass_a(qa_ref, q_ref, of_ref, pa_ref, pf_ref, w0c_ref, w0f_ref, w0i_ref,
            b0_ref, y0_ref, st_ref):
    d2 = jnp.dot(qa_ref[...], pa_ref[...],
                 preferred_element_type=jnp.float32)
    m1 = jnp.min(d2, axis=1, keepdims=True)
    t = jnp.where(d2 == m1, jnp.inf, d2)
    m2 = jnp.min(t, axis=1, keepdims=True)
    t = jnp.where(t == m2, jnp.inf, t)
    m3 = jnp.min(t, axis=1, keepdims=True)
    w = jnp.where(d2 <= m3, 1.0 / jnp.maximum(d2, 1e-12), 0.0)
    wsum = (1.0 / jnp.maximum(m1, 1e-12) + 1.0 / jnp.maximum(m2, 1e-12)
            + 1.0 / jnp.maximum(m3, 1e-12))
    interp = jnp.dot(w, pf_ref[...],
                     preferred_element_type=jnp.float32) / wsum
    y0 = (jnp.dot(q_ref[...], w0c_ref[...], precision=_HIGHEST,
                  preferred_element_type=jnp.float32)
          + jnp.dot(of_ref[...], w0f_ref[...], precision=_HIGHEST,
                    preferred_element_type=jnp.float32)
          + jnp.dot(interp, w0i_ref[...], precision=_HIGHEST,
                    preferred_element_type=jnp.float32)
          + b0_ref[...])
    y0_ref[...] = y0
    st_ref[0, 0:1, :] = jnp.sum(y0, axis=0, keepdims=True)
    st_ref[0, 1:2, :] = jnp.sum(y0 * y0, axis=0, keepdims=True)


def _make_pass_bc(n_rows, with_matmul):
    inv_n = 1.0 / n_rows

    def _stats(st_ref, g_ref, be_ref):
        part = jnp.sum(st_ref[...], axis=0)
        mean = part[0:1, :] * inv_n
        var = part[1:2, :] * inv_n - mean * mean
        scale = g_ref[...] * jax.lax.rsqrt(var + _EPS)
        shift = be_ref[...] - mean * scale
        return scale, shift

    def _pass_b(y_ref, st_ref, g_ref, be_ref, w1_ref, b1_ref, o_ref, st1_ref):
        scale, shift = _stats(st_ref, g_ref, be_ref)
        h = jnp.maximum(y_ref[...] * scale + shift, 0.0)
        y1 = jnp.dot(h, w1_ref[...], precision=_HIGHEST,
                     preferred_element_type=jnp.float32) + b1_ref[...]
        o_ref[...] = y1
        st1_ref[0, 0:1, :] = jnp.sum(y1, axis=0, keepdims=True)
        st1_ref[0, 1:2, :] = jnp.sum(y1 * y1, axis=0, keepdims=True)

    def _pass_c(y_ref, st_ref, g_ref, be_ref, o_ref):
        scale, shift = _stats(st_ref, g_ref, be_ref)
        o_ref[...] = jnp.maximum(y_ref[...] * scale + shift, 0.0)

    return _pass_b if with_matmul else _pass_c


def kernel(prop_coords, prop_feats, orig_coords, orig_feats,
           W0, b0, g0, be0, W1, b1, g1, be1):
    n_l, _ = prop_coords.shape
    n_m = orig_coords.shape[0]
    f1 = prop_feats.shape[1]
    f2 = orig_feats.shape[1]
    h = W0.shape[1]
    tm = 512
    grid = n_m // tm
    f32 = jnp.float32

    qn2 = jnp.sum(orig_coords * orig_coords, axis=1, keepdims=True)
    pn2 = jnp.sum(prop_coords * prop_coords, axis=1, keepdims=True)
    ones_q = jnp.ones((n_m, 1), f32)
    zeros_q = jnp.zeros((n_m, 3), f32)
    qa = jnp.concatenate([orig_coords, qn2, ones_q, zeros_q], axis=1)
    ones_p = jnp.ones((n_l, 1), f32)
    zeros_p = jnp.zeros((n_l, 3), f32)
    pa = jnp.concatenate([-2.0 * prop_coords, ones_p, pn2, zeros_p],
                         axis=1).T

    def _split3(x):
        hi = jax.lax.reduce_precision(x, 8, 7)
        r = x - hi
        mid = jax.lax.reduce_precision(r, 8, 7)
        lo = jax.lax.reduce_precision(r - mid, 8, 7)
        return hi, mid, lo

    qh, qm, ql = _split3(qa)
    ph, pm, plo = _split3(pa)
    qs = jnp.concatenate([qh, qh, qm, qh, ql, qm],
                         axis=1).astype(jnp.bfloat16)
    ps = jnp.concatenate([ph, pm, ph, plo, ph, pm],
                         axis=0).astype(jnp.bfloat16)
    w0c = jnp.pad(W0[:3], ((0, 5), (0, 0)))
    w0f = W0[3:3 + f2]
    w0i = W0[3 + f2:]
    b0r = b0.reshape(1, h)
    b1r = b1.reshape(1, h)
    g0r = g0.reshape(1, h)
    be0r = be0.reshape(1, h)
    g1r = g1.reshape(1, h)
    be1r = be1.reshape(1, h)

    tm_bc = 2048
    grid_bc = n_m // tm_bc
    row_spec = lambda w: pl.BlockSpec((tm, w), lambda i: (i, 0))
    row_spec_bc = pl.BlockSpec((tm_bc, h), lambda i: (i, 0))
    full = lambda shape: pl.BlockSpec(shape, lambda i: (0,) * len(shape))
    st_out_spec = pl.BlockSpec((1, 8, h), lambda i: (i, 0, 0))
    st_in_spec = pl.BlockSpec((grid, 8, h), lambda i: (0, 0, 0))
    st_out_spec_bc = pl.BlockSpec((1, 8, h), lambda i: (i, 0, 0))
    st_in_spec_bc = pl.BlockSpec((grid_bc, 8, h), lambda i: (0, 0, 0))
    st_shape = jax.ShapeDtypeStruct((grid, 8, h), f32)
    st_shape_bc = jax.ShapeDtypeStruct((grid_bc, 8, h), f32)
    params = pltpu.CompilerParams(dimension_semantics=("parallel",))

    y0, st0 = pl.pallas_call(
        _pass_a,
        grid=(grid,),
        in_specs=[row_spec(48), row_spec(8), row_spec(f2), full((48, n_l)),
                  full((n_l, f1)), full((8, h)), full((f2, h)), full((f1, h)),
                  full((1, h))],
        out_specs=[row_spec(h), st_out_spec],
        out_shape=[jax.ShapeDtypeStruct((n_m, h), f32), st_shape],
        compiler_params=params,
    )(qs, qa, orig_feats, ps, prop_feats, w0c, w0f, w0i, b0r)

    y1, st1 = pl.pallas_call(
        _make_pass_bc(n_m, True),
        grid=(grid_bc,),
        in_specs=[row_spec_bc, st_in_spec, full((1, h)), full((1, h)),
                  full((h, h)), full((1, h))],
        out_specs=[row_spec_bc, st_out_spec_bc],
        out_shape=[jax.ShapeDtypeStruct((n_m, h), f32), st_shape_bc],
        compiler_params=params,
    )(y0, st0, g0r, be0r, W1, b1r)

    out = pl.pallas_call(
        _make_pass_bc(n_m, False),
        grid=(grid_bc,),
        in_specs=[row_spec_bc, st_in_spec_bc, full((1, h)), full((1, h))],
        out_specs=row_spec_bc,
        out_shape=jax.ShapeDtypeStruct((n_m, h), f32),
        compiler_params=params,
    )(y1, st1, g1r, be1r)

    return out


# speedup vs baseline: 1.3696x; 1.0337x over previous
import jax
import jax.numpy as jnp
from jax.experimental import pallas as pl
from jax.experimental.pallas import tpu as pltpu

_HIGHEST = jax.lax.Precision.HIGHEST
_EPS = 1e-5


def _pass_a(qa_ref, q_ref, of_ref, pa_ref, pf_ref, w0c_ref, w0f_ref, w0i_ref,
            b0_ref, y0_ref, st_ref):
    d2 = jnp.dot(qa_ref[...], pa_ref[...],
                 preferred_element_type=jnp.float32)
    m1 = jnp.min(d2, axis=1, keepdims=True)
    t = jnp.where(d2 == m1, jnp.inf, d2)
    m2 = jnp.min(t, axis=1, keepdims=True)
    t = jnp.where(t == m2, jnp.inf, t)
    m3 = jnp.min(t, axis=1, keepdims=True)
    w = jnp.where(d2 <= m3, 1.0 / jnp.maximum(d2, 1e-12), 0.0)
    wsum = (1.0 / jnp.maximum(m1, 1e-12) + 1.0 / jnp.maximum(m2, 1e-12)
            + 1.0 / jnp.maximum(m3, 1e-12))
    interp = jnp.dot(w, pf_ref[...],
                     preferred_element_type=jnp.float32) / wsum
    y0 = (jnp.dot(q_ref[...], w0c_ref[...], precision=_HIGHEST,
                  preferred_element_type=jnp.float32)
          + jnp.dot(of_ref[...], w0f_ref[...], precision=_HIGHEST,
                    preferred_element_type=jnp.float32)
          + jnp.dot(interp, w0i_ref[...], precision=_HIGHEST,
                    preferred_element_type=jnp.float32)
          + b0_ref[...])
    y0_ref[...] = y0
    st_ref[0, 0:1, :] = jnp.sum(y0, axis=0, keepdims=True)
    st_ref[0, 1:2, :] = jnp.sum(y0 * y0, axis=0, keepdims=True)


def _make_pass_bc(n_rows, with_matmul):
    inv_n = 1.0 / n_rows

    def _stats(st_ref, g_ref, be_ref):
        part = jnp.sum(st_ref[...], axis=0)
        mean = part[0:1, :] * inv_n
        var = part[1:2, :] * inv_n - mean * mean
        scale = g_ref[...] * jax.lax.rsqrt(var + _EPS)
        shift = be_ref[...] - mean * scale
        return scale, shift

    def _pass_b(y_ref, st_ref, g_ref, be_ref, w1_ref, b1_ref, o_ref, st1_ref):
        scale, shift = _stats(st_ref, g_ref, be_ref)
        h = jnp.maximum(y_ref[...] * scale + shift, 0.0)
        y1 = jnp.dot(h, w1_ref[...], precision=_HIGHEST,
                     preferred_element_type=jnp.float32) + b1_ref[...]
        o_ref[...] = y1
        st1_ref[0, 0:1, :] = jnp.sum(y1, axis=0, keepdims=True)
        st1_ref[0, 1:2, :] = jnp.sum(y1 * y1, axis=0, keepdims=True)

    def _pass_c(y_ref, st_ref, g_ref, be_ref, o_ref):
        scale, shift = _stats(st_ref, g_ref, be_ref)
        o_ref[...] = jnp.maximum(y_ref[...] * scale + shift, 0.0)

    return _pass_b if with_matmul else _pass_c


def kernel(prop_coords, prop_feats, orig_coords, orig_feats,
           W0, b0, g0, be0, W1, b1, g1, be1):
    n_l, _ = prop_coords.shape
    n_m = orig_coords.shape[0]
    f1 = prop_feats.shape[1]
    f2 = orig_feats.shape[1]
    h = W0.shape[1]
    tm = 1024
    grid = n_m // tm
    f32 = jnp.float32

    qn2 = jnp.sum(orig_coords * orig_coords, axis=1, keepdims=True)
    pn2 = jnp.sum(prop_coords * prop_coords, axis=1, keepdims=True)
    ones_q = jnp.ones((n_m, 1), f32)
    zeros_q = jnp.zeros((n_m, 3), f32)
    qa = jnp.concatenate([orig_coords, qn2, ones_q, zeros_q], axis=1)
    ones_p = jnp.ones((n_l, 1), f32)
    zeros_p = jnp.zeros((n_l, 3), f32)
    pa = jnp.concatenate([-2.0 * prop_coords, ones_p, pn2, zeros_p],
                         axis=1).T

    def _split3(x):
        hi = jax.lax.reduce_precision(x, 8, 7)
        r = x - hi
        mid = jax.lax.reduce_precision(r, 8, 7)
        lo = jax.lax.reduce_precision(r - mid, 8, 7)
        return hi, mid, lo

    qh, qm, ql = _split3(qa)
    ph, pm, plo = _split3(pa)
    qs = jnp.concatenate([qh, qh, qm, qh, ql, qm],
                         axis=1).astype(jnp.bfloat16)
    ps = jnp.concatenate([ph, pm, ph, plo, ph, pm],
                         axis=0).astype(jnp.bfloat16)
    w0c = jnp.pad(W0[:3], ((0, 5), (0, 0)))
    w0f = W0[3:3 + f2]
    w0i = W0[3 + f2:]
    b0r = b0.reshape(1, h)
    b1r = b1.reshape(1, h)
    g0r = g0.reshape(1, h)
    be0r = be0.reshape(1, h)
    g1r = g1.reshape(1, h)
    be1r = be1.reshape(1, h)

    tm_bc = 2048
    grid_bc = n_m // tm_bc
    row_spec = lambda w: pl.BlockSpec((tm, w), lambda i: (i, 0))
    row_spec_bc = pl.BlockSpec((tm_bc, h), lambda i: (i, 0))
    full = lambda shape: pl.BlockSpec(shape, lambda i: (0,) * len(shape))
    st_out_spec = pl.BlockSpec((1, 8, h), lambda i: (i, 0, 0))
    st_in_spec = pl.BlockSpec((grid, 8, h), lambda i: (0, 0, 0))
    st_out_spec_bc = pl.BlockSpec((1, 8, h), lambda i: (i, 0, 0))
    st_in_spec_bc = pl.BlockSpec((grid_bc, 8, h), lambda i: (0, 0, 0))
    st_shape = jax.ShapeDtypeStruct((grid, 8, h), f32)
    st_shape_bc = jax.ShapeDtypeStruct((grid_bc, 8, h), f32)
    params = pltpu.CompilerParams(dimension_semantics=("parallel",))

    y0, st0 = pl.pallas_call(
        _pass_a,
        grid=(grid,),
        in_specs=[row_spec(48), row_spec(8), row_spec(f2), full((48, n_l)),
                  full((n_l, f1)), full((8, h)), full((f2, h)), full((f1, h)),
                  full((1, h))],
        out_specs=[row_spec(h), st_out_spec],
        out_shape=[jax.ShapeDtypeStruct((n_m, h), f32), st_shape],
        compiler_params=params,
    )(qs, qa, orig_feats, ps, prop_feats, w0c, w0f, w0i, b0r)

    y1, st1 = pl.pallas_call(
        _make_pass_bc(n_m, True),
        grid=(grid_bc,),
        in_specs=[row_spec_bc, st_in_spec, full((1, h)), full((1, h)),
                  full((h, h)), full((1, h))],
        out_specs=[row_spec_bc, st_out_spec_bc],
        out_shape=[jax.ShapeDtypeStruct((n_m, h), f32), st_shape_bc],
        compiler_params=params,
    )(y0, st0, g0r, be0r, W1, b1r)

    out = pl.pallas_call(
        _make_pass_bc(n_m, False),
        grid=(grid_bc,),
        in_specs=[row_spec_bc, st_in_spec_bc, full((1, h)), full((1, h))],
        out_specs=row_spec_bc,
        out_shape=jax.ShapeDtypeStruct((n_m, h), f32),
        compiler_params=params,
    )(y1, st1, g1r, be1r)

    return out


# speedup vs baseline: 1.3781x; 1.0062x over previous
import jax
import jax.numpy as jnp
from jax.experimental import pallas as pl
from jax.experimental.pallas import tpu as pltpu

_HIGHEST = jax.lax.Precision.HIGHEST
_EPS = 1e-5


def _pass_a(qa_ref, q_ref, of_ref, pa_ref, pf_ref, w0c_ref, w0f_ref, w0i_ref,
            b0_ref, y0_ref, st_ref):
    d2 = jnp.dot(qa_ref[...], pa_ref[...],
                 preferred_element_type=jnp.float32)
    m1 = jnp.min(d2, axis=1, keepdims=True)
    t = jnp.where(d2 == m1, jnp.inf, d2)
    m2 = jnp.min(t, axis=1, keepdims=True)
    t = jnp.where(t == m2, jnp.inf, t)
    m3 = jnp.min(t, axis=1, keepdims=True)
    w = jnp.where(d2 <= m3, 1.0 / jnp.maximum(d2, 1e-12), 0.0)
    wsum = (1.0 / jnp.maximum(m1, 1e-12) + 1.0 / jnp.maximum(m2, 1e-12)
            + 1.0 / jnp.maximum(m3, 1e-12))
    interp = jnp.dot(w, pf_ref[...],
                     preferred_element_type=jnp.float32) / wsum
    y0 = (jnp.dot(q_ref[...], w0c_ref[...], precision=_HIGHEST,
                  preferred_element_type=jnp.float32)
          + jnp.dot(of_ref[...], w0f_ref[...], precision=_HIGHEST,
                    preferred_element_type=jnp.float32)
          + jnp.dot(interp, w0i_ref[...], precision=_HIGHEST,
                    preferred_element_type=jnp.float32)
          + b0_ref[...])
    y0_ref[...] = y0
    st_ref[0, 0:1, :] = jnp.sum(y0, axis=0, keepdims=True)
    st_ref[0, 1:2, :] = jnp.sum(y0 * y0, axis=0, keepdims=True)


def _make_pass_bc(n_rows, with_matmul):
    inv_n = 1.0 / n_rows

    def _stats(st_ref, g_ref, be_ref):
        part = jnp.sum(st_ref[...], axis=0)
        mean = part[0:1, :] * inv_n
        var = part[1:2, :] * inv_n - mean * mean
        scale = g_ref[...] * jax.lax.rsqrt(var + _EPS)
        shift = be_ref[...] - mean * scale
        return scale, shift

    def _pass_b(y_ref, st_ref, g_ref, be_ref, w1_ref, b1_ref, o_ref, st1_ref):
        scale, shift = _stats(st_ref, g_ref, be_ref)
        h = jnp.maximum(y_ref[...] * scale + shift, 0.0)
        y1 = jnp.dot(h, w1_ref[...], precision=_HIGHEST,
                     preferred_element_type=jnp.float32) + b1_ref[...]
        o_ref[...] = y1
        st1_ref[0, 0:1, :] = jnp.sum(y1, axis=0, keepdims=True)
        st1_ref[0, 1:2, :] = jnp.sum(y1 * y1, axis=0, keepdims=True)

    def _pass_c(y_ref, st_ref, g_ref, be_ref, o_ref):
        scale, shift = _stats(st_ref, g_ref, be_ref)
        o_ref[...] = jnp.maximum(y_ref[...] * scale + shift, 0.0)

    return _pass_b if with_matmul else _pass_c


def kernel(prop_coords, prop_feats, orig_coords, orig_feats,
           W0, b0, g0, be0, W1, b1, g1, be1):
    n_l, _ = prop_coords.shape
    n_m = orig_coords.shape[0]
    f1 = prop_feats.shape[1]
    f2 = orig_feats.shape[1]
    h = W0.shape[1]
    tm = 2048
    grid = n_m // tm
    f32 = jnp.float32

    qn2 = jnp.sum(orig_coords * orig_coords, axis=1, keepdims=True)
    pn2 = jnp.sum(prop_coords * prop_coords, axis=1, keepdims=True)
    ones_q = jnp.ones((n_m, 1), f32)
    zeros_q = jnp.zeros((n_m, 3), f32)
    qa = jnp.concatenate([orig_coords, qn2, ones_q, zeros_q], axis=1)
    ones_p = jnp.ones((n_l, 1), f32)
    zeros_p = jnp.zeros((n_l, 3), f32)
    pa = jnp.concatenate([-2.0 * prop_coords, ones_p, pn2, zeros_p],
                         axis=1).T

    def _split3(x):
        hi = jax.lax.reduce_precision(x, 8, 7)
        r = x - hi
        mid = jax.lax.reduce_precision(r, 8, 7)
        lo = jax.lax.reduce_precision(r - mid, 8, 7)
        return hi, mid, lo

    qh, qm, ql = _split3(qa)
    ph, pm, plo = _split3(pa)
    qs = jnp.concatenate([qh, qh, qm, qh, ql, qm],
                         axis=1).astype(jnp.bfloat16)
    ps = jnp.concatenate([ph, pm, ph, plo, ph, pm],
                         axis=0).astype(jnp.bfloat16)
    w0c = jnp.pad(W0[:3], ((0, 5), (0, 0)))
    w0f = W0[3:3 + f2]
    w0i = W0[3 + f2:]
    b0r = b0.reshape(1, h)
    b1r = b1.reshape(1, h)
    g0r = g0.reshape(1, h)
    be0r = be0.reshape(1, h)
    g1r = g1.reshape(1, h)
    be1r = be1.reshape(1, h)

    tm_bc = 2048
    grid_bc = n_m // tm_bc
    row_spec = lambda w: pl.BlockSpec((tm, w), lambda i: (i, 0))
    row_spec_bc = pl.BlockSpec((tm_bc, h), lambda i: (i, 0))
    full = lambda shape: pl.BlockSpec(shape, lambda i: (0,) * len(shape))
    st_out_spec = pl.BlockSpec((1, 8, h), lambda i: (i, 0, 0))
    st_in_spec = pl.BlockSpec((grid, 8, h), lambda i: (0, 0, 0))
    st_out_spec_bc = pl.BlockSpec((1, 8, h), lambda i: (i, 0, 0))
    st_in_spec_bc = pl.BlockSpec((grid_bc, 8, h), lambda i: (0, 0, 0))
    st_shape = jax.ShapeDtypeStruct((grid, 8, h), f32)
    st_shape_bc = jax.ShapeDtypeStruct((grid_bc, 8, h), f32)
    params = pltpu.CompilerParams(dimension_semantics=("parallel",))

    y0, st0 = pl.pallas_call(
        _pass_a,
        grid=(grid,),
        in_specs=[row_spec(48), row_spec(8), row_spec(f2), full((48, n_l)),
                  full((n_l, f1)), full((8, h)), full((f2, h)), full((f1, h)),
                  full((1, h))],
        out_specs=[row_spec(h), st_out_spec],
        out_shape=[jax.ShapeDtypeStruct((n_m, h), f32), st_shape],
        compiler_params=params,
    )(qs, qa, orig_feats, ps, prop_feats, w0c, w0f, w0i, b0r)

    y1, st1 = pl.pallas_call(
        _make_pass_bc(n_m, True),
        grid=(grid_bc,),
        in_specs=[row_spec_bc, st_in_spec, full((1, h)), full((1, h)),
                  full((h, h)), full((1, h))],
        out_specs=[row_spec_bc, st_out_spec_bc],
        out_shape=[jax.ShapeDtypeStruct((n_m, h), f32), st_shape_bc],
        compiler_params=params,
    )(y0, st0, g0r, be0r, W1, b1r)

    out = pl.pallas_call(
        _make_pass_bc(n_m, False),
        grid=(grid_bc,),
        in_specs=[row_spec_bc, st_in_spec_bc, full((1, h)), full((1, h))],
        out_specs=row_spec_bc,
        out_shape=jax.ShapeDtypeStruct((n_m, h), f32),
        compiler_params=params,
    )(y1, st1, g1r, be1r)

    return out
